# Initial kernel scaffold; baseline (speedup 1.0000x reference)
import jax
import jax.numpy as jnp
from jax.experimental import pallas as pl


def kernel(x, Wr, br, W1, b1, g1, beta1, W2, b2):
    raise NotImplementedError("write your pallas kernel here")


# speedup vs baseline: 4.1893x; 4.1893x over previous
import functools
import math

import jax
import jax.numpy as jnp
from jax.experimental import pallas as pl

_E = 8
_D = 768
_H = 256
_EPS_LN = 1e-5
_BT = 512
_EPAD = 128

_SQRT2 = math.sqrt(2.0)


def _moe_body(x_ref, wr_ref, br_ref, w1_ref, b1_ref, g1_ref, beta1_ref,
              w2_ref, b2_ref, out_ref):
    xb = x_ref[...]
    logits = jnp.dot(xb, wr_ref[...], preferred_element_type=jnp.float32)
    logits = logits + br_ref[...]
    eio = jax.lax.broadcasted_iota(jnp.int32, (_BT, _EPAD), 1)

    m1 = jnp.max(logits, axis=-1, keepdims=True)
    e1 = jnp.min(jnp.where(logits == m1, eio, _EPAD), axis=-1, keepdims=True)
    l2 = jnp.where(eio == e1, -jnp.inf, logits)
    m2 = jnp.max(l2, axis=-1, keepdims=True)
    e2 = jnp.min(jnp.where(l2 == m2, eio, _EPAD), axis=-1, keepdims=True)
    wa = jax.nn.sigmoid(m1 - m2)
    wb = 1.0 - wa

    xbf = xb.astype(jnp.bfloat16)
    acc = jnp.zeros((_BT, _D), jnp.float32)
    for e in range(_E):
        we = jnp.where(e1 == e, wa, 0.0) + jnp.where(e2 == e, wb, 0.0)
        h = jnp.dot(xbf, w1_ref[e], preferred_element_type=jnp.float32)
        h = h + b1_ref[e:e + 1, :]
        mu = jnp.mean(h, axis=-1, keepdims=True)
        var = jnp.mean((h - mu) ** 2, axis=-1, keepdims=True)
        hn = (h - mu) * jax.lax.rsqrt(var + _EPS_LN)
        hn = hn * g1_ref[e:e + 1, :] + beta1_ref[e:e + 1, :]
        a = 0.5 * hn * (1.0 + jax.lax.erf(hn / _SQRT2))
        aw = (a * we).astype(jnp.bfloat16)
        acc = acc + jnp.dot(aw, w2_ref[e], preferred_element_type=jnp.float32)
        acc = acc + we * b2_ref[e:e + 1, :]
    out_ref[...] = acc


def kernel(x, Wr, br, W1, b1, g1, beta1, W2, b2):
    orig_shape = x.shape
    n = orig_shape[0] * orig_shape[1]
    x2 = x.reshape(n, _D)
    wr_p = jnp.zeros((_D, _EPAD), jnp.float32).at[:, :_E].set(Wr)
    br_p = jnp.full((1, _EPAD), -jnp.inf, jnp.float32).at[0, :_E].set(br)
    w1_bf = W1.astype(jnp.bfloat16)
    w2_bf = W2.astype(jnp.bfloat16)

    grid = (n // _BT,)
    y = pl.pallas_call(
        _moe_body,
        grid=grid,
        in_specs=[
            pl.BlockSpec((_BT, _D), lambda i: (i, 0)),
            pl.BlockSpec((_D, _EPAD), lambda i: (0, 0)),
            pl.BlockSpec((1, _EPAD), lambda i: (0, 0)),
            pl.BlockSpec((_E, _D, _H), lambda i: (0, 0, 0)),
            pl.BlockSpec((_E, _H), lambda i: (0, 0)),
            pl.BlockSpec((_E, _H), lambda i: (0, 0)),
            pl.BlockSpec((_E, _H), lambda i: (0, 0)),
            pl.BlockSpec((_E, _H, _D), lambda i: (0, 0, 0)),
            pl.BlockSpec((_E, _D), lambda i: (0, 0)),
        ],
        out_specs=pl.BlockSpec((_BT, _D), lambda i: (i, 0)),
        out_shape=jax.ShapeDtypeStruct((n, _D), jnp.float32),
    )(x2, wr_p, br_p, w1_bf, b1, g1, beta1, w2_bf, b2)
    return y.reshape(orig_shape)
